# baseline (speedup 1.0000x reference)
import functools

import jax
import jax.numpy as jnp
from jax.experimental import pallas as pl
from jax.experimental.pallas import tpu as pltpu

_VMEM_LIMIT = 48 * 1024 * 1024


def _b16(x):
    return x.astype(jnp.bfloat16)


def _dot(x, w):
    return jnp.dot(x, w, preferred_element_type=jnp.float32)


def _mm(x, w_ref, b_ref):
    return _dot(_b16(x), w_ref[...]) + b_ref[...]


def _w(a):
    return pl.BlockSpec(a.shape, lambda *_: (0,) * a.ndim)


def _tile(n, target):
    if n <= target:
        return n
    t = target - (target % 8)
    while t >= 8:
        if n % t == 0:
            return t
        t -= 8
    return n


def _conv_kernel(x_ref, w_ref, b_ref, o_ref):
    y = jax.lax.dot_general(x_ref[0], w_ref[...],
                            (((0,), (0,)), ((), ())),
                            preferred_element_type=jnp.float32) + b_ref[...]
    o_ref[0] = pltpu.bitcast(y.astype(jnp.bfloat16), jnp.int32)


def _psp(img_chw, w, b, tile=8192):
    B, Cin, P = img_chw.shape
    Cout = w.shape[1]
    tp = _tile(P, tile)
    return pl.pallas_call(
        _conv_kernel,
        out_shape=jax.ShapeDtypeStruct((B, P // 2, Cout), jnp.int32),
        grid=(B, P // tp),
        in_specs=[pl.BlockSpec((1, Cin, tp), lambda b_, p_: (b_, 0, p_)),
                  _w(w), _w(b)],
        out_specs=pl.BlockSpec((1, tp // 2, Cout), lambda b_, p_: (b_, p_, 0)),
        compiler_params=pltpu.CompilerParams(
            dimension_semantics=("parallel", "parallel"),
            vmem_limit_bytes=_VMEM_LIMIT),
    )(img_chw, w, b)


def _inst_kernel(inv_n, tn, choose_ref, featp_ref, pm_ref, pts_ref,
                 gw1, gb1, gw2, gb2, gw3p, cwp, fb,
                 iw1, ib1, iw2, ib2,
                 local_ref, global_ref, gscr):
    n_idx = pl.program_id(1)

    def gather(i, _):
        idx = choose_ref[0, 0, 0, i]
        gscr[i, :] = featp_ref[0, idx >> 1, :]
        return 0

    jax.lax.fori_loop(0, tn, gather, 0, unroll=8)
    e2 = pltpu.bitcast(gscr[...], jnp.bfloat16)
    e2 = e2 * pm_ref[0]
    ec = _dot(e2, cwp[...])
    ec = jnp.sum(ec.reshape(tn, 2, ec.shape[-1]), axis=1)

    h = jnp.maximum(_dot(pts_ref[0], gw1[...]) + gb1[...], 0.0)
    h = jnp.maximum(_mm(h, gw2, gb2), 0.0)
    inst_local = jnp.maximum(
        _dot(_b16(h), gw3p[...]) + ec + fb[...], 0.0)
    local_ref[0] = inst_local.astype(local_ref.dtype)
    g = jnp.maximum(_mm(inst_local, iw1, ib1), 0.0)
    g = jnp.maximum(_mm(g, iw2, ib2), 0.0)
    tile_sum = jnp.sum(g, axis=0, keepdims=True)

    @pl.when(n_idx == 0)
    def _():
        global_ref[0] = jnp.zeros_like(global_ref[0])

    global_ref[0] += tile_sum

    @pl.when(n_idx == pl.num_programs(1) - 1)
    def _():
        global_ref[0] *= inv_n


def _instance(featp, choose, pm, points, geo, ig, tile=2048):
    B, N, _ = points.shape
    P2 = featp.shape[1]
    tn = _tile(N, tile)
    kern = functools.partial(_inst_kernel, 1.0 / float(N), tn)
    return pl.pallas_call(
        kern,
        out_shape=(jax.ShapeDtypeStruct((B, N, 128), jnp.bfloat16),
                   jax.ShapeDtypeStruct((B, 1, 1024), jnp.float32)),
        grid=(B, N // tn),
        in_specs=[
            pl.BlockSpec((1, 1, 1, tn), lambda b, n: (b, n, 0, 0),
                         memory_space=pltpu.SMEM),
            pl.BlockSpec((1, P2, 32), lambda b, n: (b, 0, 0)),
            pl.BlockSpec((1, 2 * tn, 1), lambda b, n: (b, n, 0)),
            pl.BlockSpec((1, tn, 3), lambda b, n: (b, n, 0)),
            _w(geo['w1']), _w(geo['b1']), _w(geo['w2']), _w(geo['b2']),
            _w(geo['w3p']), _w(geo['cwp']), _w(geo['fb']),
            _w(ig['w1']), _w(ig['b1']), _w(ig['w2']), _w(ig['b2']),
        ],
        out_specs=(pl.BlockSpec((1, tn, 128), lambda b, n: (b, n, 0)),
                   pl.BlockSpec((1, 1, 1024), lambda b, n: (b, 0, 0))),
        scratch_shapes=[pltpu.VMEM((tn, 32), jnp.int32)],
        compiler_params=pltpu.CompilerParams(
            dimension_semantics=("parallel", "arbitrary"),
            vmem_limit_bytes=60000 * 1024),
    )(choose.reshape(B, N // tn, 1, tn), featp, pm, points,
      geo['w1'], geo['b1'], geo['w2'], geo['b2'],
      geo['w3p'], geo['cwp'], geo['fb'],
      ig['w1'], ig['b1'], ig['w2'], ig['b2'])


def _deform_kernel(inv_nv, B, NV, CH,
                   prior_ref, ig_ref, mask_ref, fold_ref, pool_ref,
                   lw1, lb1, lw2, lb2, lw3, lb3,
                   gw1, gb1, gw2, gb2,
                   dw1, dwig, dwcg, db1, dw2, db2, w3, b3,
                   awig, awcg,
                   abias_ref, acc_ref, cl_scr, cg_scr):
    s = pl.program_id(0)
    n_chunks = B // CH
    rows = CH * NV

    @pl.when(s == 0)
    def _():
        acc_ref[...] = jnp.zeros_like(acc_ref)

    def pass1(c):
        r0 = c * rows
        pr = prior_ref[pl.ds(c * CH, CH)] + acc_ref[pl.ds(c * CH, CH)]
        x = pr.reshape(rows, 3)
        h = jnp.maximum(_dot(x, lw1[...]) + lb1[...], 0.0)
        h = jnp.maximum(_dot(_b16(h), lw2[...]) + lb2[...], 0.0)
        clc = jnp.maximum(_dot(_b16(h), lw3[...]) + lb3[...], 0.0)
        cl_scr[pl.ds(r0, rows)] = _b16(clc)
        g = jnp.maximum(_dot(_b16(clc), gw1[...]) + gb1[...], 0.0)
        g = jnp.maximum(_dot(_b16(g), gw2[...]) + gb2[...], 0.0)
        cg_scr[pl.ds(c * CH, CH)] = (
            jnp.sum(g.reshape(CH, NV, g.shape[-1]), axis=1) * inv_nv)

    for c in range(n_chunks):
        pass1(c)

    ig_b = _b16(ig_ref[:, 0, :])
    cg_b = _b16(cg_scr[...])
    bias_d = _dot(ig_b, dwig[0]) + _dot(cg_b, dwcg[0])

    def pass2(c):
        r0 = c * rows
        clc = cl_scr[pl.ds(r0, rows)]
        bd = bias_d[c * CH:(c + 1) * CH]
        bd = jnp.broadcast_to(bd[:, None, :], (CH, NV, bd.shape[1])
                              ).reshape(rows, bd.shape[1])
        h1 = jnp.maximum(_dot(clc, dw1[0]) + db1[0] + bd, 0.0)
        h2 = jnp.maximum(_dot(_b16(h1), dw2[0]) + db2[0], 0.0)
        dall = _dot(_b16(h2), w3[0]) + b3[0]
        m = mask_ref[c * CH:(c + 1) * CH]
        m = jnp.broadcast_to(m[:, None, :], (CH, NV, m.shape[1])
                             ).reshape(rows, m.shape[1])
        delta = jnp.dot(dall * m, fold_ref[...],
                        preferred_element_type=jnp.float32)
        acc_ref[pl.ds(c * CH, CH)] = (acc_ref[pl.ds(c * CH, CH)]
                                      + delta.reshape(CH, NV, 3))

    for c in range(n_chunks):
        pass2(c)

    @pl.when(s == pl.num_programs(0) - 1)
    def _():
        abias_ref[:, 0, :] = _dot(ig_b, awig[...]) + _dot(cg_b, awcg[...])


def _deform(prior, inst_global, cat_id, cl, cg, d, a):
    B, NV, _ = prior.shape
    n_stage, n_cat, k3, cout = d['w3'].shape
    CH = 4 if B % 4 == 0 else 1
    w3s = jnp.transpose(d['w3'], (0, 2, 1, 3)).reshape(n_stage, k3, n_cat * cout)
    b3s = jnp.transpose(d['b3'], (0, 2, 1, 3)).reshape(n_stage, 1, n_cat * cout)
    onehot = jax.nn.one_hot(cat_id, n_cat, dtype=jnp.float32)
    mask18 = jnp.repeat(onehot, cout, axis=1)
    fold = jnp.tile(jnp.eye(cout, dtype=jnp.float32), (n_cat, 1))
    pool = jnp.kron(jnp.eye(CH, dtype=jnp.float32),
                    jnp.full((1, NV), 1.0 / float(NV), jnp.float32))
    kern = functools.partial(_deform_kernel, 1.0 / float(NV), B, NV, CH)
    return pl.pallas_call(
        kern,
        out_shape=(jax.ShapeDtypeStruct((B, 1, 512), jnp.float32),
                   jax.ShapeDtypeStruct((B, NV, 3), jnp.float32)),
        grid=(n_stage,),
        in_specs=[
            _w(prior), _w(inst_global), _w(mask18), _w(fold), _w(pool),
            _w(cl['w1']), _w(cl['b1']), _w(cl['w2']), _w(cl['b2']),
            _w(cl['w3']), _w(cl['b3']),
            _w(cg['w1']), _w(cg['b1']), _w(cg['w2']), _w(cg['b2']),
            pl.BlockSpec((1,) + d['w1_loc'].shape[1:], lambda s: (s, 0, 0)),
            pl.BlockSpec((1,) + d['w1_ig'].shape[1:], lambda s: (s, 0, 0)),
            pl.BlockSpec((1,) + d['w1_cg'].shape[1:], lambda s: (s, 0, 0)),
            pl.BlockSpec((1,) + d['b1'].shape[1:], lambda s: (s, 0, 0)),
            pl.BlockSpec((1,) + d['w2'].shape[1:], lambda s: (s, 0, 0)),
            pl.BlockSpec((1,) + d['b2'].shape[1:], lambda s: (s, 0, 0)),
            pl.BlockSpec((1, k3, n_cat * cout), lambda s: (s, 0, 0)),
            pl.BlockSpec((1, 1, n_cat * cout), lambda s: (s, 0, 0)),
            _w(a['w1_ig']), _w(a['w1_cg']),
        ],
        out_specs=[
            pl.BlockSpec((B, 1, 512), lambda s: (0, 0, 0)),
            pl.BlockSpec((B, NV, 3), lambda s: (0, 0, 0)),
        ],
        scratch_shapes=[pltpu.VMEM((B * NV, 64), jnp.bfloat16),
                        pltpu.VMEM((B, 1024), jnp.float32)],
        compiler_params=pltpu.CompilerParams(
            dimension_semantics=("arbitrary",),
            vmem_limit_bytes=60000 * 1024),
    )(prior, inst_global, mask18, fold, pool,
      cl['w1'], cl['b1'], cl['w2'], cl['b2'], cl['w3'], cl['b3'],
      cg['w1'], cg['b1'], cg['w2'], cg['b2'],
      d['w1_loc'], d['w1_ig'], d['w1_cg'], d['b1'], d['w2'], d['b2'],
      w3s, b3s, a['w1_ig'], a['w1_cg'])


def _assign_kernel(cat_ref, x_ref, bg_ref,
                   w1, b1, w2, b2, w3, b3, o_ref):
    del cat_ref
    bias1 = bg_ref[0] + b1[...]
    h1 = jnp.maximum(_dot(x_ref[0], w1[...]) + bias1, 0.0)
    h2 = jnp.maximum(_mm(h1, w2, b2), 0.0)
    y = _dot(_b16(h2), w3[0]) + b3[0]
    o_ref[0] = y.astype(o_ref.dtype)


def _assign(x_local, assign_bias, cat_id, p, tile=2048):
    B, N, Cloc = x_local.shape
    n_cat, k3, cout = p['w3'].shape
    tn = _tile(N, tile)
    grid_spec = pltpu.PrefetchScalarGridSpec(
        num_scalar_prefetch=1,
        grid=(B, N // tn),
        in_specs=[
            pl.BlockSpec((1, tn, Cloc), lambda b, n, cat: (b, n, 0)),
            pl.BlockSpec((1, 1, 512), lambda b, n, cat: (b, 0, 0)),
            _w(p['w1_loc']), _w(p['b1']), _w(p['w2']), _w(p['b2']),
            pl.BlockSpec((1, k3, cout), lambda b, n, cat: (cat[b], 0, 0)),
            pl.BlockSpec((1, 1, cout), lambda b, n, cat: (cat[b], 0, 0)),
        ],
        out_specs=pl.BlockSpec((1, tn, cout), lambda b, n, cat: (b, n, 0)),
    )
    return pl.pallas_call(
        _assign_kernel,
        out_shape=jax.ShapeDtypeStruct((B, N, cout), jnp.float32),
        grid_spec=grid_spec,
        compiler_params=pltpu.CompilerParams(
            dimension_semantics=("parallel", "parallel"),
            vmem_limit_bytes=_VMEM_LIMIT),
    )(cat_id, x_local, assign_bias, p['w1_loc'], p['b1'], p['w2'], p['b2'],
      p['w3'], p['b3'])


def kernel(points, img, choose, cat_id, prior, nocs, model,
           psp_w, psp_b,
           ig_w1, ig_b1, ig_w2, ig_b2, ig_w3p, ig_cwp, ig_fb,
           cl_w1, cl_b1, cl_w2, cl_b2, cl_w3, cl_b3,
           igl_w1, igl_b1, igl_w2, igl_b2,
           cgl_w1, cgl_b1, cgl_w2, cgl_b2,
           a_w1_loc, a_w1_ig, a_w1_cg, a_b1, a_w2, a_b2, a_w3, a_b3,
           d_w1_loc, d_w1_ig, d_w1_cg, d_b1, d_w2, d_b2, d_w3, d_b3):
    del nocs, model
    B, C, H, W = img.shape

    featp = _psp(img.reshape(B, C, H * W).astype(jnp.bfloat16), psp_w, psp_b)
    N = choose.shape[1]
    par = (choose & 1).astype(jnp.bfloat16)
    pm = jnp.stack([1.0 - par, par], axis=2).reshape(B, 2 * N, 1)

    geo = dict(w1=ig_w1, b1=ig_b1, w2=ig_w2, b2=ig_b2,
               w3p=ig_w3p, cwp=ig_cwp, fb=ig_fb)
    igl = dict(w1=igl_w1, b1=igl_b1, w2=igl_w2, b2=igl_b2)
    inst_local, inst_global = _instance(featp, choose, pm, points, geo, igl)

    cl = dict(w1=cl_w1, b1=cl_b1, w2=cl_w2, b2=cl_b2, w3=cl_w3, b3=cl_b3)
    cgl = dict(w1=cgl_w1, b1=cgl_b1, w2=cgl_w2, b2=cgl_b2)
    d = dict(w1_loc=d_w1_loc, w1_ig=d_w1_ig, w1_cg=d_w1_cg, b1=d_b1,
             w2=d_w2, b2=d_b2, w3=d_w3, b3=d_b3)
    a = dict(w1_loc=a_w1_loc, w1_ig=a_w1_ig, w1_cg=a_w1_cg, b1=a_b1,
             w2=a_w2, b2=a_b2, w3=a_w3, b3=a_b3)
    assign_bias, deltas_acc = _deform(prior, inst_global, cat_id, cl, cgl, d, a)

    assign_mat = _assign(inst_local, assign_bias, cat_id, a)

    zero = jnp.float32(0.0)
    return assign_mat, deltas_acc, zero, zero, zero, zero, zero


# speedup vs baseline: 1.0405x; 1.0405x over previous
import functools

import jax
import jax.numpy as jnp
from jax.experimental import pallas as pl
from jax.experimental.pallas import tpu as pltpu

_VMEM_LIMIT = 48 * 1024 * 1024


def _b16(x):
    return x.astype(jnp.bfloat16)


def _dot(x, w):
    return jnp.dot(x, w, preferred_element_type=jnp.float32)


def _mm(x, w_ref, b_ref):
    return _dot(_b16(x), w_ref[...]) + b_ref[...]


def _w(a):
    return pl.BlockSpec(a.shape, lambda *_: (0,) * a.ndim)


def _tile(n, target):
    if n <= target:
        return n
    t = target - (target % 8)
    while t >= 8:
        if n % t == 0:
            return t
        t -= 8
    return n


def _conv_kernel(x_ref, w_ref, b_ref, o_ref):
    y = jax.lax.dot_general(x_ref[0], w_ref[...],
                            (((0,), (0,)), ((), ())),
                            preferred_element_type=jnp.float32) + b_ref[...]
    o_ref[0] = pltpu.bitcast(y.astype(jnp.bfloat16), jnp.int32)


def _psp(img_chw, w, b, tile=8192):
    B, Cin, P = img_chw.shape
    Cout = w.shape[1]
    tp = _tile(P, tile)
    return pl.pallas_call(
        _conv_kernel,
        out_shape=jax.ShapeDtypeStruct((B, P // 2, Cout), jnp.int32),
        grid=(B, P // tp),
        in_specs=[pl.BlockSpec((1, Cin, tp), lambda b_, p_: (b_, 0, p_)),
                  _w(w), _w(b)],
        out_specs=pl.BlockSpec((1, tp // 2, Cout), lambda b_, p_: (b_, p_, 0)),
        compiler_params=pltpu.CompilerParams(
            dimension_semantics=("parallel", "parallel"),
            vmem_limit_bytes=_VMEM_LIMIT),
    )(img_chw, w, b)


def _inst_kernel(inv_n, tn, choose_ref, featp_ref, pm_ref, pts_ref,
                 gw1, gb1, gw2, gb2, gw3p, cwp, fb,
                 iw1, ib1, iw2, ib2,
                 local_ref, global_ref, gscr):
    n_idx = pl.program_id(1)

    def gather(i, _):
        idx = choose_ref[0, 0, 0, i]
        gscr[i, :] = featp_ref[0, idx >> 1, :]
        return 0

    jax.lax.fori_loop(0, tn, gather, 0, unroll=32)
    e2 = pltpu.bitcast(gscr[...], jnp.bfloat16)
    e2 = e2 * pm_ref[0]
    ec = _dot(e2, cwp[...])
    ec = jnp.sum(ec.reshape(tn, 2, ec.shape[-1]), axis=1)

    h = jnp.maximum(_dot(pts_ref[0], gw1[...]) + gb1[...], 0.0)
    h = jnp.maximum(_mm(h, gw2, gb2), 0.0)
    inst_local = jnp.maximum(
        _dot(_b16(h), gw3p[...]) + ec + fb[...], 0.0)
    local_ref[0] = inst_local.astype(local_ref.dtype)
    g = jnp.maximum(_mm(inst_local, iw1, ib1), 0.0)
    g = jnp.maximum(_mm(g, iw2, ib2), 0.0)
    tile_sum = jnp.sum(g, axis=0, keepdims=True)

    @pl.when(n_idx == 0)
    def _():
        global_ref[0] = jnp.zeros_like(global_ref[0])

    global_ref[0] += tile_sum

    @pl.when(n_idx == pl.num_programs(1) - 1)
    def _():
        global_ref[0] *= inv_n


def _instance(featp, choose, pm, points, geo, ig, tile=2048):
    B, N, _ = points.shape
    P2 = featp.shape[1]
    tn = _tile(N, tile)
    kern = functools.partial(_inst_kernel, 1.0 / float(N), tn)
    return pl.pallas_call(
        kern,
        out_shape=(jax.ShapeDtypeStruct((B, N, 128), jnp.bfloat16),
                   jax.ShapeDtypeStruct((B, 1, 1024), jnp.float32)),
        grid=(B, N // tn),
        in_specs=[
            pl.BlockSpec((1, 1, 1, tn), lambda b, n: (b, n, 0, 0),
                         memory_space=pltpu.SMEM),
            pl.BlockSpec((1, P2, 32), lambda b, n: (b, 0, 0)),
            pl.BlockSpec((1, 2 * tn, 1), lambda b, n: (b, n, 0)),
            pl.BlockSpec((1, tn, 3), lambda b, n: (b, n, 0)),
            _w(geo['w1']), _w(geo['b1']), _w(geo['w2']), _w(geo['b2']),
            _w(geo['w3p']), _w(geo['cwp']), _w(geo['fb']),
            _w(ig['w1']), _w(ig['b1']), _w(ig['w2']), _w(ig['b2']),
        ],
        out_specs=(pl.BlockSpec((1, tn, 128), lambda b, n: (b, n, 0)),
                   pl.BlockSpec((1, 1, 1024), lambda b, n: (b, 0, 0))),
        scratch_shapes=[pltpu.VMEM((tn, 32), jnp.int32)],
        compiler_params=pltpu.CompilerParams(
            dimension_semantics=("parallel", "arbitrary"),
            vmem_limit_bytes=60000 * 1024),
    )(choose.reshape(B, N // tn, 1, tn), featp, pm, points,
      geo['w1'], geo['b1'], geo['w2'], geo['b2'],
      geo['w3p'], geo['cwp'], geo['fb'],
      ig['w1'], ig['b1'], ig['w2'], ig['b2'])


def _deform_kernel(inv_nv, B, NV, CH,
                   prior_ref, ig_ref, mask_ref, fold_ref, pool_ref,
                   lw1, lb1, lw2, lb2, lw3, lb3,
                   gw1, gb1, gw2, gb2,
                   dw1, dwig, dwcg, db1, dw2, db2, w3, b3,
                   awig, awcg,
                   abias_ref, acc_ref, cl_scr, cg_scr):
    s = pl.program_id(0)
    n_chunks = B // CH
    rows = CH * NV

    @pl.when(s == 0)
    def _():
        acc_ref[...] = jnp.zeros_like(acc_ref)

    def pass1(c):
        r0 = c * rows
        pr = prior_ref[pl.ds(c * CH, CH)] + acc_ref[pl.ds(c * CH, CH)]
        x = pr.reshape(rows, 3)
        h = jnp.maximum(_dot(x, lw1[...]) + lb1[...], 0.0)
        h = jnp.maximum(_dot(_b16(h), lw2[...]) + lb2[...], 0.0)
        clc = jnp.maximum(_dot(_b16(h), lw3[...]) + lb3[...], 0.0)
        cl_scr[pl.ds(r0, rows)] = _b16(clc)
        g = jnp.maximum(_dot(_b16(clc), gw1[...]) + gb1[...], 0.0)
        g = jnp.maximum(_dot(_b16(g), gw2[...]) + gb2[...], 0.0)
        cg_scr[pl.ds(c * CH, CH)] = (
            jnp.sum(g.reshape(CH, NV, g.shape[-1]), axis=1) * inv_nv)

    for c in range(n_chunks):
        pass1(c)

    ig_b = _b16(ig_ref[:, 0, :])
    cg_b = _b16(cg_scr[...])
    bias_d = _dot(ig_b, dwig[0]) + _dot(cg_b, dwcg[0])

    def pass2(c):
        r0 = c * rows
        clc = cl_scr[pl.ds(r0, rows)]
        bd = bias_d[c * CH:(c + 1) * CH]
        bd = jnp.broadcast_to(bd[:, None, :], (CH, NV, bd.shape[1])
                              ).reshape(rows, bd.shape[1])
        h1 = jnp.maximum(_dot(clc, dw1[0]) + db1[0] + bd, 0.0)
        h2 = jnp.maximum(_dot(_b16(h1), dw2[0]) + db2[0], 0.0)
        dall = _dot(_b16(h2), w3[0]) + b3[0]
        m = mask_ref[c * CH:(c + 1) * CH]
        m = jnp.broadcast_to(m[:, None, :], (CH, NV, m.shape[1])
                             ).reshape(rows, m.shape[1])
        delta = jnp.dot(dall * m, fold_ref[...],
                        preferred_element_type=jnp.float32)
        acc_ref[pl.ds(c * CH, CH)] = (acc_ref[pl.ds(c * CH, CH)]
                                      + delta.reshape(CH, NV, 3))

    for c in range(n_chunks):
        pass2(c)

    @pl.when(s == pl.num_programs(0) - 1)
    def _():
        abias_ref[:, 0, :] = _dot(ig_b, awig[...]) + _dot(cg_b, awcg[...])


def _deform(prior, inst_global, cat_id, cl, cg, d, a):
    B, NV, _ = prior.shape
    n_stage, n_cat, k3, cout = d['w3'].shape
    CH = 4 if B % 4 == 0 else 1
    w3s = jnp.transpose(d['w3'], (0, 2, 1, 3)).reshape(n_stage, k3, n_cat * cout)
    b3s = jnp.transpose(d['b3'], (0, 2, 1, 3)).reshape(n_stage, 1, n_cat * cout)
    onehot = jax.nn.one_hot(cat_id, n_cat, dtype=jnp.float32)
    mask18 = jnp.repeat(onehot, cout, axis=1)
    fold = jnp.tile(jnp.eye(cout, dtype=jnp.float32), (n_cat, 1))
    pool = jnp.kron(jnp.eye(CH, dtype=jnp.float32),
                    jnp.full((1, NV), 1.0 / float(NV), jnp.float32))
    kern = functools.partial(_deform_kernel, 1.0 / float(NV), B, NV, CH)
    return pl.pallas_call(
        kern,
        out_shape=(jax.ShapeDtypeStruct((B, 1, 512), jnp.float32),
                   jax.ShapeDtypeStruct((B, NV, 3), jnp.float32)),
        grid=(n_stage,),
        in_specs=[
            _w(prior), _w(inst_global), _w(mask18), _w(fold), _w(pool),
            _w(cl['w1']), _w(cl['b1']), _w(cl['w2']), _w(cl['b2']),
            _w(cl['w3']), _w(cl['b3']),
            _w(cg['w1']), _w(cg['b1']), _w(cg['w2']), _w(cg['b2']),
            pl.BlockSpec((1,) + d['w1_loc'].shape[1:], lambda s: (s, 0, 0)),
            pl.BlockSpec((1,) + d['w1_ig'].shape[1:], lambda s: (s, 0, 0)),
            pl.BlockSpec((1,) + d['w1_cg'].shape[1:], lambda s: (s, 0, 0)),
            pl.BlockSpec((1,) + d['b1'].shape[1:], lambda s: (s, 0, 0)),
            pl.BlockSpec((1,) + d['w2'].shape[1:], lambda s: (s, 0, 0)),
            pl.BlockSpec((1,) + d['b2'].shape[1:], lambda s: (s, 0, 0)),
            pl.BlockSpec((1, k3, n_cat * cout), lambda s: (s, 0, 0)),
            pl.BlockSpec((1, 1, n_cat * cout), lambda s: (s, 0, 0)),
            _w(a['w1_ig']), _w(a['w1_cg']),
        ],
        out_specs=[
            pl.BlockSpec((B, 1, 512), lambda s: (0, 0, 0)),
            pl.BlockSpec((B, NV, 3), lambda s: (0, 0, 0)),
        ],
        scratch_shapes=[pltpu.VMEM((B * NV, 64), jnp.bfloat16),
                        pltpu.VMEM((B, 1024), jnp.float32)],
        compiler_params=pltpu.CompilerParams(
            dimension_semantics=("arbitrary",),
            vmem_limit_bytes=60000 * 1024),
    )(prior, inst_global, mask18, fold, pool,
      cl['w1'], cl['b1'], cl['w2'], cl['b2'], cl['w3'], cl['b3'],
      cg['w1'], cg['b1'], cg['w2'], cg['b2'],
      d['w1_loc'], d['w1_ig'], d['w1_cg'], d['b1'], d['w2'], d['b2'],
      w3s, b3s, a['w1_ig'], a['w1_cg'])


def _assign_kernel(cat_ref, x_ref, bg_ref,
                   w1, b1, w2, b2, w3, b3, o_ref):
    del cat_ref
    bias1 = bg_ref[0] + b1[...]
    h1 = jnp.maximum(_dot(x_ref[0], w1[...]) + bias1, 0.0)
    h2 = jnp.maximum(_mm(h1, w2, b2), 0.0)
    y = _dot(_b16(h2), w3[0]) + b3[0]
    o_ref[0] = y.astype(o_ref.dtype)


def _assign(x_local, assign_bias, cat_id, p, tile=2048):
    B, N, Cloc = x_local.shape
    n_cat, k3, cout = p['w3'].shape
    tn = _tile(N, tile)
    grid_spec = pltpu.PrefetchScalarGridSpec(
        num_scalar_prefetch=1,
        grid=(B, N // tn),
        in_specs=[
            pl.BlockSpec((1, tn, Cloc), lambda b, n, cat: (b, n, 0)),
            pl.BlockSpec((1, 1, 512), lambda b, n, cat: (b, 0, 0)),
            _w(p['w1_loc']), _w(p['b1']), _w(p['w2']), _w(p['b2']),
            pl.BlockSpec((1, k3, cout), lambda b, n, cat: (cat[b], 0, 0)),
            pl.BlockSpec((1, 1, cout), lambda b, n, cat: (cat[b], 0, 0)),
        ],
        out_specs=pl.BlockSpec((1, tn, cout), lambda b, n, cat: (b, n, 0)),
    )
    return pl.pallas_call(
        _assign_kernel,
        out_shape=jax.ShapeDtypeStruct((B, N, cout), jnp.float32),
        grid_spec=grid_spec,
        compiler_params=pltpu.CompilerParams(
            dimension_semantics=("parallel", "parallel"),
            vmem_limit_bytes=_VMEM_LIMIT),
    )(cat_id, x_local, assign_bias, p['w1_loc'], p['b1'], p['w2'], p['b2'],
      p['w3'], p['b3'])


def kernel(points, img, choose, cat_id, prior, nocs, model,
           psp_w, psp_b,
           ig_w1, ig_b1, ig_w2, ig_b2, ig_w3p, ig_cwp, ig_fb,
           cl_w1, cl_b1, cl_w2, cl_b2, cl_w3, cl_b3,
           igl_w1, igl_b1, igl_w2, igl_b2,
           cgl_w1, cgl_b1, cgl_w2, cgl_b2,
           a_w1_loc, a_w1_ig, a_w1_cg, a_b1, a_w2, a_b2, a_w3, a_b3,
           d_w1_loc, d_w1_ig, d_w1_cg, d_b1, d_w2, d_b2, d_w3, d_b3):
    del nocs, model
    B, C, H, W = img.shape

    featp = _psp(img.reshape(B, C, H * W).astype(jnp.bfloat16), psp_w, psp_b)
    N = choose.shape[1]
    par = (choose & 1).astype(jnp.bfloat16)
    pm = jnp.stack([1.0 - par, par], axis=2).reshape(B, 2 * N, 1)

    geo = dict(w1=ig_w1, b1=ig_b1, w2=ig_w2, b2=ig_b2,
               w3p=ig_w3p, cwp=ig_cwp, fb=ig_fb)
    igl = dict(w1=igl_w1, b1=igl_b1, w2=igl_w2, b2=igl_b2)
    inst_local, inst_global = _instance(featp, choose, pm, points, geo, igl)

    cl = dict(w1=cl_w1, b1=cl_b1, w2=cl_w2, b2=cl_b2, w3=cl_w3, b3=cl_b3)
    cgl = dict(w1=cgl_w1, b1=cgl_b1, w2=cgl_w2, b2=cgl_b2)
    d = dict(w1_loc=d_w1_loc, w1_ig=d_w1_ig, w1_cg=d_w1_cg, b1=d_b1,
             w2=d_w2, b2=d_b2, w3=d_w3, b3=d_b3)
    a = dict(w1_loc=a_w1_loc, w1_ig=a_w1_ig, w1_cg=a_w1_cg, b1=a_b1,
             w2=a_w2, b2=a_b2, w3=a_w3, b3=a_b3)
    assign_bias, deltas_acc = _deform(prior, inst_global, cat_id, cl, cgl, d, a)

    assign_mat = _assign(inst_local, assign_bias, cat_id, a)

    zero = jnp.float32(0.0)
    return assign_mat, deltas_acc, zero, zero, zero, zero, zero


# speedup vs baseline: 1.0479x; 1.0071x over previous
import functools

import jax
import jax.numpy as jnp
from jax.experimental import pallas as pl
from jax.experimental.pallas import tpu as pltpu

_VMEM_LIMIT = 48 * 1024 * 1024


def _b16(x):
    return x.astype(jnp.bfloat16)


def _dot(x, w):
    return jnp.dot(x, w, preferred_element_type=jnp.float32)


def _mm(x, w_ref, b_ref):
    return _dot(_b16(x), w_ref[...]) + b_ref[...]


def _w(a):
    return pl.BlockSpec(a.shape, lambda *_: (0,) * a.ndim)


def _tile(n, target):
    if n <= target:
        return n
    t = target - (target % 8)
    while t >= 8:
        if n % t == 0:
            return t
        t -= 8
    return n


def _conv_kernel(x_ref, w_ref, b_ref, o_ref):
    y = jax.lax.dot_general(x_ref[0], w_ref[...],
                            (((0,), (0,)), ((), ())),
                            preferred_element_type=jnp.float32) + b_ref[...]
    o_ref[0] = y.astype(o_ref.dtype)


def _psp(img_chw, w, b, tile=8192):
    B, Cin, P = img_chw.shape
    Cout = w.shape[1]
    tp = _tile(P, tile)
    return pl.pallas_call(
        _conv_kernel,
        out_shape=jax.ShapeDtypeStruct((B, P, Cout), jnp.bfloat16),
        grid=(B, P // tp),
        in_specs=[pl.BlockSpec((1, Cin, tp), lambda b_, p_: (b_, 0, p_)),
                  _w(w), _w(b)],
        out_specs=pl.BlockSpec((1, tp, Cout), lambda b_, p_: (b_, p_, 0)),
        compiler_params=pltpu.CompilerParams(
            dimension_semantics=("parallel", "parallel"),
            vmem_limit_bytes=_VMEM_LIMIT),
    )(img_chw, w, b)


def _inst_kernel(inv_n, emb_ref, pts_ref,
                 gw1, gb1, gw2, gb2, gw3p, cwp, fb,
                 iw1, ib1, iw2, ib2,
                 local_ref, global_ref):
    n_idx = pl.program_id(1)
    emb = emb_ref[0]
    h = jnp.maximum(_dot(pts_ref[0], gw1[...]) + gb1[...], 0.0)
    h = jnp.maximum(_mm(h, gw2, gb2), 0.0)
    inst_local = jnp.maximum(
        _dot(_b16(h), gw3p[...]) + _dot(emb, cwp[...]) + fb[...], 0.0)
    local_ref[0] = inst_local.astype(local_ref.dtype)
    g = jnp.maximum(_mm(inst_local, iw1, ib1), 0.0)
    g = jnp.maximum(_mm(g, iw2, ib2), 0.0)
    tile_sum = jnp.sum(g, axis=0, keepdims=True)

    @pl.when(n_idx == 0)
    def _():
        global_ref[0] = jnp.zeros_like(global_ref[0])

    global_ref[0] += tile_sum

    @pl.when(n_idx == pl.num_programs(1) - 1)
    def _():
        global_ref[0] *= inv_n


def _instance(emb, points, geo, ig, tile=4096):
    B, N, _ = points.shape
    tn = _tile(N, tile)
    kern = functools.partial(_inst_kernel, 1.0 / float(N))
    return pl.pallas_call(
        kern,
        out_shape=(jax.ShapeDtypeStruct((B, N, 128), jnp.bfloat16),
                   jax.ShapeDtypeStruct((B, 1, 1024), jnp.float32)),
        grid=(B, N // tn),
        in_specs=[
            pl.BlockSpec((1, tn, 32), lambda b, n: (b, n, 0)),
            pl.BlockSpec((1, tn, 3), lambda b, n: (b, n, 0)),
            _w(geo['w1']), _w(geo['b1']), _w(geo['w2']), _w(geo['b2']),
            _w(geo['w3p']), _w(geo['cwp']), _w(geo['fb']),
            _w(ig['w1']), _w(ig['b1']), _w(ig['w2']), _w(ig['b2']),
        ],
        out_specs=(pl.BlockSpec((1, tn, 128), lambda b, n: (b, n, 0)),
                   pl.BlockSpec((1, 1, 1024), lambda b, n: (b, 0, 0))),
        compiler_params=pltpu.CompilerParams(
            dimension_semantics=("parallel", "arbitrary"),
            vmem_limit_bytes=_VMEM_LIMIT),
    )(emb, points,
      geo['w1'], geo['b1'], geo['w2'], geo['b2'],
      geo['w3p'], geo['cwp'], geo['fb'],
      ig['w1'], ig['b1'], ig['w2'], ig['b2'])


def _deform_kernel(inv_nv, B, NV, CH,
                   prior_ref, ig_ref, mask_ref, fold_ref,
                   lw1, lb1, lw2, lb2, lw3, lb3,
                   gw1, gb1, gw2, gb2,
                   dw1, dwig, dwcg, db1, dw2, db2, w3, b3,
                   awig, awcg,
                   abias_ref, acc_ref, cl_scr, cg_scr):
    s = pl.program_id(0)
    n_chunks = B // CH
    rows = CH * NV

    @pl.when(s == 0)
    def _():
        acc_ref[...] = jnp.zeros_like(acc_ref)

    def pass1(c):
        r0 = c * rows
        pr = prior_ref[pl.ds(c * CH, CH)] + acc_ref[pl.ds(c * CH, CH)]
        x = pr.reshape(rows, 3)
        h = jnp.maximum(_dot(x, lw1[...]) + lb1[...], 0.0)
        h = jnp.maximum(_dot(_b16(h), lw2[...]) + lb2[...], 0.0)
        clc = jnp.maximum(_dot(_b16(h), lw3[...]) + lb3[...], 0.0)
        cl_scr[pl.ds(r0, rows)] = _b16(clc)
        g = jnp.maximum(_dot(_b16(clc), gw1[...]) + gb1[...], 0.0)
        g = jnp.maximum(_dot(_b16(g), gw2[...]) + gb2[...], 0.0)
        cg_scr[pl.ds(c * CH, CH)] = (
            jnp.sum(g.reshape(CH, NV, g.shape[-1]), axis=1) * inv_nv)

    for c in range(n_chunks):
        pass1(c)

    ig_b = _b16(ig_ref[:, 0, :])
    cg_b = _b16(cg_scr[...])
    bias_d = _dot(ig_b, dwig[0]) + _dot(cg_b, dwcg[0])

    def pass2(c):
        r0 = c * rows
        clc = cl_scr[pl.ds(r0, rows)]
        bd = bias_d[c * CH:(c + 1) * CH]
        bd = jnp.broadcast_to(bd[:, None, :], (CH, NV, bd.shape[1])
                              ).reshape(rows, bd.shape[1])
        h1 = jnp.maximum(_dot(clc, dw1[0]) + db1[0] + bd, 0.0)
        h2 = jnp.maximum(_dot(_b16(h1), dw2[0]) + db2[0], 0.0)
        dall = _dot(_b16(h2), w3[0]) + b3[0]
        m = mask_ref[c * CH:(c + 1) * CH]
        m = jnp.broadcast_to(m[:, None, :], (CH, NV, m.shape[1])
                             ).reshape(rows, m.shape[1])
        delta = jnp.dot(dall * m, fold_ref[...],
                        preferred_element_type=jnp.float32)
        acc_ref[pl.ds(c * CH, CH)] = (acc_ref[pl.ds(c * CH, CH)]
                                      + delta.reshape(CH, NV, 3))

    for c in range(n_chunks):
        pass2(c)

    @pl.when(s == pl.num_programs(0) - 1)
    def _():
        abias_ref[:, 0, :] = _dot(ig_b, awig[...]) + _dot(cg_b, awcg[...])


def _deform(prior, inst_global, cat_id, cl, cg, d, a):
    B, NV, _ = prior.shape
    n_stage, n_cat, k3, cout = d['w3'].shape
    CH = 4 if B % 4 == 0 else 1
    w3s = jnp.transpose(d['w3'], (0, 2, 1, 3)).reshape(n_stage, k3, n_cat * cout)
    b3s = jnp.transpose(d['b3'], (0, 2, 1, 3)).reshape(n_stage, 1, n_cat * cout)
    onehot = jax.nn.one_hot(cat_id, n_cat, dtype=jnp.float32)
    mask18 = jnp.repeat(onehot, cout, axis=1)
    fold = jnp.tile(jnp.eye(cout, dtype=jnp.float32), (n_cat, 1))
    kern = functools.partial(_deform_kernel, 1.0 / float(NV), B, NV, CH)
    return pl.pallas_call(
        kern,
        out_shape=(jax.ShapeDtypeStruct((B, 1, 512), jnp.float32),
                   jax.ShapeDtypeStruct((B, NV, 3), jnp.float32)),
        grid=(n_stage,),
        in_specs=[
            _w(prior), _w(inst_global), _w(mask18), _w(fold),
            _w(cl['w1']), _w(cl['b1']), _w(cl['w2']), _w(cl['b2']),
            _w(cl['w3']), _w(cl['b3']),
            _w(cg['w1']), _w(cg['b1']), _w(cg['w2']), _w(cg['b2']),
            pl.BlockSpec((1,) + d['w1_loc'].shape[1:], lambda s: (s, 0, 0)),
            pl.BlockSpec((1,) + d['w1_ig'].shape[1:], lambda s: (s, 0, 0)),
            pl.BlockSpec((1,) + d['w1_cg'].shape[1:], lambda s: (s, 0, 0)),
            pl.BlockSpec((1,) + d['b1'].shape[1:], lambda s: (s, 0, 0)),
            pl.BlockSpec((1,) + d['w2'].shape[1:], lambda s: (s, 0, 0)),
            pl.BlockSpec((1,) + d['b2'].shape[1:], lambda s: (s, 0, 0)),
            pl.BlockSpec((1, k3, n_cat * cout), lambda s: (s, 0, 0)),
            pl.BlockSpec((1, 1, n_cat * cout), lambda s: (s, 0, 0)),
            _w(a['w1_ig']), _w(a['w1_cg']),
        ],
        out_specs=[
            pl.BlockSpec((B, 1, 512), lambda s: (0, 0, 0)),
            pl.BlockSpec((B, NV, 3), lambda s: (0, 0, 0)),
        ],
        scratch_shapes=[pltpu.VMEM((B * NV, 64), jnp.bfloat16),
                        pltpu.VMEM((B, 1024), jnp.float32)],
        compiler_params=pltpu.CompilerParams(
            dimension_semantics=("arbitrary",),
            vmem_limit_bytes=60000 * 1024),
    )(prior, inst_global, mask18, fold,
      cl['w1'], cl['b1'], cl['w2'], cl['b2'], cl['w3'], cl['b3'],
      cg['w1'], cg['b1'], cg['w2'], cg['b2'],
      d['w1_loc'], d['w1_ig'], d['w1_cg'], d['b1'], d['w2'], d['b2'],
      w3s, b3s, a['w1_ig'], a['w1_cg'])


def _assign_kernel(cat_ref, x_ref, bg_ref,
                   w1, b1, w2, b2, w3, b3, o_ref):
    del cat_ref
    bias1 = bg_ref[0] + b1[...]
    h1 = jnp.maximum(_dot(x_ref[0], w1[...]) + bias1, 0.0)
    h2 = jnp.maximum(_mm(h1, w2, b2), 0.0)
    y = _dot(_b16(h2), w3[0]) + b3[0]
    o_ref[0] = y.astype(o_ref.dtype)


def _assign(x_local, assign_bias, cat_id, p, tile=2048):
    B, N, Cloc = x_local.shape
    n_cat, k3, cout = p['w3'].shape
    tn = _tile(N, tile)
    grid_spec = pltpu.PrefetchScalarGridSpec(
        num_scalar_prefetch=1,
        grid=(B, N // tn),
        in_specs=[
            pl.BlockSpec((1, tn, Cloc), lambda b, n, cat: (b, n, 0)),
            pl.BlockSpec((1, 1, 512), lambda b, n, cat: (b, 0, 0)),
            _w(p['w1_loc']), _w(p['b1']), _w(p['w2']), _w(p['b2']),
            pl.BlockSpec((1, k3, cout), lambda b, n, cat: (cat[b], 0, 0)),
            pl.BlockSpec((1, 1, cout), lambda b, n, cat: (cat[b], 0, 0)),
        ],
        out_specs=pl.BlockSpec((1, tn, cout), lambda b, n, cat: (b, n, 0)),
    )
    return pl.pallas_call(
        _assign_kernel,
        out_shape=jax.ShapeDtypeStruct((B, N, cout), jnp.float32),
        grid_spec=grid_spec,
        compiler_params=pltpu.CompilerParams(
            dimension_semantics=("parallel", "parallel"),
            vmem_limit_bytes=_VMEM_LIMIT),
    )(cat_id, x_local, assign_bias, p['w1_loc'], p['b1'], p['w2'], p['b2'],
      p['w3'], p['b3'])


def kernel(points, img, choose, cat_id, prior, nocs, model,
           psp_w, psp_b,
           ig_w1, ig_b1, ig_w2, ig_b2, ig_w3p, ig_cwp, ig_fb,
           cl_w1, cl_b1, cl_w2, cl_b2, cl_w3, cl_b3,
           igl_w1, igl_b1, igl_w2, igl_b2,
           cgl_w1, cgl_b1, cgl_w2, cgl_b2,
           a_w1_loc, a_w1_ig, a_w1_cg, a_b1, a_w2, a_b2, a_w3, a_b3,
           d_w1_loc, d_w1_ig, d_w1_cg, d_b1, d_w2, d_b2, d_w3, d_b3):
    del nocs, model
    B, C, H, W = img.shape

    feat = _psp(img.reshape(B, C, H * W).astype(jnp.bfloat16), psp_w, psp_b)
    N = choose.shape[1]
    flat_idx = (choose + jnp.arange(B, dtype=choose.dtype)[:, None] * (H * W)).reshape(-1)
    emb = feat.reshape(B * H * W, 32).at[flat_idx].get(
        mode="promise_in_bounds").reshape(B, N, 32)

    geo = dict(w1=ig_w1, b1=ig_b1, w2=ig_w2, b2=ig_b2,
               w3p=ig_w3p, cwp=ig_cwp, fb=ig_fb)
    igl = dict(w1=igl_w1, b1=igl_b1, w2=igl_w2, b2=igl_b2)
    inst_local, inst_global = _instance(emb, points, geo, igl)

    cl = dict(w1=cl_w1, b1=cl_b1, w2=cl_w2, b2=cl_b2, w3=cl_w3, b3=cl_b3)
    cgl = dict(w1=cgl_w1, b1=cgl_b1, w2=cgl_w2, b2=cgl_b2)
    d = dict(w1_loc=d_w1_loc, w1_ig=d_w1_ig, w1_cg=d_w1_cg, b1=d_b1,
             w2=d_w2, b2=d_b2, w3=d_w3, b3=d_b3)
    a = dict(w1_loc=a_w1_loc, w1_ig=a_w1_ig, w1_cg=a_w1_cg, b1=a_b1,
             w2=a_w2, b2=a_b2, w3=a_w3, b3=a_b3)
    assign_bias, deltas_acc = _deform(prior, inst_global, cat_id, cl, cgl, d, a)

    assign_mat = _assign(inst_local, assign_bias, cat_id, a)

    zero = jnp.float32(0.0)
    return assign_mat, deltas_acc, zero, zero, zero, zero, zero
